# Initial kernel scaffold; baseline (speedup 1.0000x reference)
import jax
import jax.numpy as jnp
from jax.experimental import pallas as pl


def kernel(logits, labels, attention_mask):
    raise NotImplementedError("write your pallas kernel here")


# speedup vs baseline: 42.1413x; 42.1413x over previous
import functools

import jax
import jax.numpy as jnp
from jax.experimental import pallas as pl
from jax.experimental.pallas import tpu as pltpu

_MUSIC_LO = 100
_MUSIC_HI = 132
_N_SPECIAL = 3
_PENALTY = 100.0
_TOP_K = 5
_ROW_BLOCK = 8


def _body(x_ref, lab_ref, am_ref, tot_ref, ce_ref, pen_ref, cnt_ref, acc_ref,
          *, n_blocks, n_rows, vocab):
    i = pl.program_id(0)

    @pl.when(i == 0)
    def _init():
        acc_ref[0] = 0.0
        acc_ref[1] = 0.0
        acc_ref[2] = 0.0
        acc_ref[3] = 0.0

    x = x_ref[...]
    lab = lab_ref[0]
    am = am_ref[0]
    col = jax.lax.broadcasted_iota(jnp.int32, (_ROW_BLOCK, vocab), 1)

    valid = lab != -100
    slab = jnp.where(valid, lab, 0)

    rmax = jnp.max(x, axis=1, keepdims=True)
    sexp = jnp.sum(jnp.exp(x - rmax), axis=1, keepdims=True)
    lse = jnp.log(sexp) + rmax
    lab_logit = jnp.sum(jnp.where(col == slab, x, 0.0), axis=1, keepdims=True)
    nll = (lse - lab_logit) * valid.astype(jnp.float32)

    xc = x
    vals = []
    nonmus = []
    for _ in range(_TOP_K):
        m = jnp.max(xc, axis=1, keepdims=True)
        ism = xc == m
        idx = jnp.min(jnp.where(ism, col, vocab), axis=1, keepdims=True)
        music = (idx < _N_SPECIAL) | ((idx >= _MUSIC_LO) & (idx < _MUSIC_HI))
        vals.append(m)
        nonmus.append(~music)
        xc = jnp.where(col == idx, -jnp.inf, xc)

    exps = [jnp.exp(v - vals[0]) for v in vals]
    esum = exps[0]
    for e in exps[1:]:
        esum = esum + e
    pmax = jnp.zeros_like(esum)
    any_nm = jnp.zeros_like(valid)
    for e, nm in zip(exps, nonmus):
        pmax = jnp.maximum(pmax, jnp.where(nm, e, 0.0))
        any_nm = any_nm | nm
    pmax = jnp.maximum(pmax / esum, 1e-12)
    pp = any_nm & (am == 1) & valid
    ppf = pp.astype(jnp.float32)
    pen = -jnp.log(pmax) * ppf * _PENALTY

    acc_ref[0] = acc_ref[0] + jnp.sum(nll)
    acc_ref[1] = acc_ref[1] + jnp.sum(valid.astype(jnp.float32))
    acc_ref[2] = acc_ref[2] + jnp.sum(pen)
    acc_ref[3] = acc_ref[3] + jnp.sum(ppf)

    @pl.when(i == n_blocks - 1)
    def _fin():
        ce = acc_ref[0] / jnp.maximum(acc_ref[1], 1.0)
        pl_ = acc_ref[2] / n_rows
        tot_ref[0] = ce + pl_
        ce_ref[0] = ce
        pen_ref[0] = pl_
        cnt_ref[0] = acc_ref[3].astype(jnp.int32)


def kernel(logits, labels, attention_mask):
    b, s, vocab = logits.shape
    n_rows = b * s
    n_blocks = n_rows // _ROW_BLOCK

    x = logits.reshape(n_rows, vocab)
    lab3 = labels.reshape(n_blocks, _ROW_BLOCK, 1)
    am3 = attention_mask.reshape(n_blocks, _ROW_BLOCK, 1)

    body = functools.partial(_body, n_blocks=n_blocks, n_rows=float(n_rows),
                             vocab=vocab)
    smem_out = pl.BlockSpec(memory_space=pltpu.SMEM)
    tot, ce, pen, cnt = pl.pallas_call(
        body,
        grid=(n_blocks,),
        in_specs=[
            pl.BlockSpec((_ROW_BLOCK, vocab), lambda i: (i, 0)),
            pl.BlockSpec((1, _ROW_BLOCK, 1), lambda i: (i, 0, 0)),
            pl.BlockSpec((1, _ROW_BLOCK, 1), lambda i: (i, 0, 0)),
        ],
        out_specs=[smem_out, smem_out, smem_out, smem_out],
        out_shape=[
            jax.ShapeDtypeStruct((1,), jnp.float32),
            jax.ShapeDtypeStruct((1,), jnp.float32),
            jax.ShapeDtypeStruct((1,), jnp.float32),
            jax.ShapeDtypeStruct((1,), jnp.int32),
        ],
        scratch_shapes=[pltpu.SMEM((4,), jnp.float32)],
    )(x, lab3, am3)
    return (tot[0], ce[0], pen[0], cnt[0])
